# baseline (speedup 1.0000x reference)
import functools

import jax
import jax.numpy as jnp
from jax import lax
from jax.experimental import pallas as pl
from jax.experimental.pallas import tpu as pltpu
from jax.experimental.pallas import tpu_sc as plsc

NC = 2
NS = 16
NW = NC * NS
CH = 128


def _emb_kernel(B, D, n_chunks):
    mesh = plsc.VectorSubcoreMesh(core_axis_name="c", subcore_axis_name="s")
    D2 = 2 * D

    @functools.partial(
        pl.kernel,
        mesh=mesh,
        compiler_params=pltpu.CompilerParams(needs_layout_passes=False),
        out_type=jax.ShapeDtypeStruct((B, D), jnp.float32),
        scratch_types=[
            pltpu.VMEM((n_chunks, CH), jnp.int32),
            pltpu.VMEM((2, CH), jnp.int32),
            pltpu.VMEM((2, CH, D2), jnp.float32),
            pltpu.VMEM((2, CH, D), jnp.float32),
            pltpu.SemaphoreType.DMA,
            pltpu.SemaphoreType.DMA,
            pltpu.SemaphoreType.DMA,
        ],
    )
    def k(vidx_hbm, px_hbm, tab_hbm, out_hbm,
          idx_v, px_v, buf, outb, gsem, psem, osem):
        wid = lax.axis_index("s") * NC + lax.axis_index("c")
        base = wid * (n_chunks * CH)
        pltpu.sync_copy(vidx_hbm.at[wid], idx_v)

        def start_chunk(j, slot):
            pltpu.async_copy(tab_hbm.at[idx_v.at[j]], buf.at[slot], gsem)
            pltpu.async_copy(px_hbm.at[wid, j], px_v.at[slot], psem)

        def finish_chunk(j, slot):
            pltpu.make_async_copy(tab_hbm.at[idx_v.at[j]], buf.at[slot], gsem).wait()
            pltpu.make_async_copy(px_hbm.at[wid, j], px_v.at[slot], psem).wait()
            lanes = lax.iota(jnp.int32, 16)
            slot16 = jnp.full((16,), slot, jnp.int32)

            def scale_row(r, _):
                r16 = jnp.full((16,), r, jnp.int32)
                p16 = plsc.load_gather(px_v, [slot16, r16])
                off = p16 * D + lanes
                for c in range(D // 16):
                    v = plsc.load_gather(buf, [slot16, r16, off + c * 16])
                    outb[slot, r, pl.ds(c * 16, 16)] = v * 8.0
                return 0

            lax.fori_loop(0, CH, scale_row, 0, unroll=2)
            pltpu.async_copy(
                outb.at[slot], out_hbm.at[pl.ds(base + j * CH, CH)], osem
            )

        def drain_out(j, slot):
            pltpu.make_async_copy(
                outb.at[slot], out_hbm.at[pl.ds(base + j * CH, CH)], osem
            ).wait()

        start_chunk(0, 0)

        def body(j, _):
            slot = lax.rem(j, 2)
            nslot = lax.rem(j + 1, 2)

            @pl.when(j + 1 < n_chunks)
            def _():
                start_chunk(j + 1, nslot)

            @pl.when(j >= 2)
            def _():
                drain_out(j - 2, slot)

            finish_chunk(j, slot)
            return 0

        lax.fori_loop(0, n_chunks, body, 0)
        drain_out(n_chunks - 2, lax.rem(n_chunks - 2, 2))
        drain_out(n_chunks - 1, lax.rem(n_chunks - 1, 2))

    return k


def kernel(x, table):
    S0, S1 = x.shape
    V, D = table.shape
    B = S0 * S1
    n_chunks = B // (NW * CH)
    xi = x.astype(jnp.int32)
    vidx = (xi >> 1).reshape(NW, n_chunks, CH)
    px = (xi & 1).reshape(NW, n_chunks, CH)
    tabv = table.reshape(V // 2, 2 * D)
    out = _emb_kernel(B, D, n_chunks)(vidx, px, tabv)
    return out.reshape(S0, S1, D)
